# baseline (speedup 1.0000x reference)
import functools
import math

import jax
import jax.numpy as jnp
from jax import lax
from jax.experimental import pallas as pl
from jax.experimental.pallas import tpu as pltpu
from jax.experimental.pallas import tpu_sc as plsc

_N = 200
_B = 10
_L = 16
_STD0 = 0.62177
_STD1 = 2.0
_LOG_STD0 = math.log(_STD0)
_LOG_STD1 = math.log(_STD1)
_HALF_LOG_2PI = 0.5 * math.log(2.0 * math.pi)
_SCALE = float(_N) / float(_B)


def _body(x_hbm, m1_hbm, ind_hbm, data_hbm, out_hbm,
          x_v, m1_v, ind_v, data_v, out_v):
    cid = lax.axis_index("c")
    sid = lax.axis_index("s")

    @pl.when(jnp.logical_and(cid == 0, sid == 0))
    def _():
        pltpu.sync_copy(x_hbm, x_v.at[pl.ds(0, 1)])
        pltpu.sync_copy(m1_hbm, m1_v.at[pl.ds(0, 1)])
        pltpu.sync_copy(ind_hbm, ind_v.at[pl.ds(0, _B)])
        pltpu.sync_copy(data_hbm, data_v)

        lane = lax.iota(jnp.int32, _L)
        mask = lane < _B
        idx = jnp.where(mask, ind_v[...], jnp.zeros((_L,), jnp.int32))
        vals = plsc.load_gather(data_v, [idx])
        xs = x_v[...][0]
        m1s = m1_v[...][0]

        branch0 = m1s < 0.5
        std = jnp.where(branch0, jnp.float32(_STD0), jnp.float32(_STD1))
        log_std = jnp.where(branch0, jnp.float32(_LOG_STD0),
                            jnp.float32(_LOG_STD1))
        z = (vals - xs) / std
        lp = -0.5 * z * z - log_std - jnp.float32(_HALF_LOG_2PI)
        ll = jnp.sum(jnp.where(mask, lp, jnp.float32(0.0)))
        prior_x = -0.5 * xs * xs - jnp.float32(_HALF_LOG_2PI)
        total = prior_x + jnp.float32(_SCALE) * ll

        out_v[...] = jnp.broadcast_to(total, (_L,))
        pltpu.sync_copy(out_v.at[pl.ds(0, 1)], out_hbm)


@jax.jit
def _log_joint(x, m1, ind, data):
    mesh = plsc.VectorSubcoreMesh(core_axis_name="c", subcore_axis_name="s")
    return pl.kernel(
        _body,
        out_type=jax.ShapeDtypeStruct((1,), jnp.float32),
        mesh=mesh,
        compiler_params=pltpu.CompilerParams(needs_layout_passes=False),
        scratch_types=[
            pltpu.VMEM((_L,), jnp.float32),
            pltpu.VMEM((_L,), jnp.float32),
            pltpu.VMEM((_L,), jnp.int32),
            pltpu.VMEM((_N,), jnp.float32),
            pltpu.VMEM((_L,), jnp.float32),
        ],
    )(x, m1, ind, data)


def kernel(x, m1, ind, data):
    return _log_joint(x, m1, ind, data)[0]


# speedup vs baseline: 1.1336x; 1.1336x over previous
import functools
import math

import jax
import jax.numpy as jnp
from jax import lax
from jax.experimental import pallas as pl
from jax.experimental.pallas import tpu as pltpu
from jax.experimental.pallas import tpu_sc as plsc

_N = 200
_B = 10
_L = 16
_STD0 = 0.62177
_STD1 = 2.0
_LOG_STD0 = math.log(_STD0)
_LOG_STD1 = math.log(_STD1)
_HALF_LOG_2PI = 0.5 * math.log(2.0 * math.pi)
_SCALE = float(_N) / float(_B)


def _body(x_hbm, m1_hbm, ind_hbm, data_hbm, out_hbm,
          x_v, m1_v, ind_v, data_v, out_v, sem):
    cid = lax.axis_index("c")
    sid = lax.axis_index("s")

    @pl.when(jnp.logical_and(cid == 0, sid == 0))
    def _():
        c1 = pltpu.make_async_copy(x_hbm, x_v.at[pl.ds(0, 1)], sem)
        c2 = pltpu.make_async_copy(m1_hbm, m1_v.at[pl.ds(0, 1)], sem)
        c3 = pltpu.make_async_copy(ind_hbm, ind_v.at[pl.ds(0, _B)], sem)
        c4 = pltpu.make_async_copy(data_hbm, data_v, sem)
        c1.start()
        c2.start()
        c3.start()
        c4.start()
        c1.wait()
        c2.wait()
        c3.wait()
        c4.wait()

        lane = lax.iota(jnp.int32, _L)
        mask = lane < _B
        idx = jnp.where(mask, ind_v[...], jnp.zeros((_L,), jnp.int32))
        vals = plsc.load_gather(data_v, [idx])
        xs = x_v[...][0]
        m1s = m1_v[...][0]

        branch0 = m1s < 0.5
        std = jnp.where(branch0, jnp.float32(_STD0), jnp.float32(_STD1))
        log_std = jnp.where(branch0, jnp.float32(_LOG_STD0),
                            jnp.float32(_LOG_STD1))
        z = (vals - xs) / std
        lp = -0.5 * z * z - log_std - jnp.float32(_HALF_LOG_2PI)
        ll = jnp.sum(jnp.where(mask, lp, jnp.float32(0.0)))
        prior_x = -0.5 * xs * xs - jnp.float32(_HALF_LOG_2PI)
        total = prior_x + jnp.float32(_SCALE) * ll

        out_v[...] = jnp.broadcast_to(total, (_L,))
        pltpu.sync_copy(out_v.at[pl.ds(0, 1)], out_hbm)


@jax.jit
def _log_joint(x, m1, ind, data):
    mesh = plsc.VectorSubcoreMesh(core_axis_name="c", subcore_axis_name="s",
                                  num_cores=1, num_subcores=1)
    return pl.kernel(
        _body,
        out_type=jax.ShapeDtypeStruct((1,), jnp.float32),
        mesh=mesh,
        compiler_params=pltpu.CompilerParams(needs_layout_passes=False),
        scratch_types=[
            pltpu.VMEM((_L,), jnp.float32),
            pltpu.VMEM((_L,), jnp.float32),
            pltpu.VMEM((_L,), jnp.int32),
            pltpu.VMEM((_N,), jnp.float32),
            pltpu.VMEM((_L,), jnp.float32),
            pltpu.SemaphoreType.DMA,
        ],
    )(x, m1, ind, data)


def kernel(x, m1, ind, data):
    return _log_joint(x, m1, ind, data)[0]


# speedup vs baseline: 1.1400x; 1.0056x over previous
import functools
import math

import jax
import jax.numpy as jnp
from jax import lax
from jax.experimental import pallas as pl
from jax.experimental.pallas import tpu as pltpu
from jax.experimental.pallas import tpu_sc as plsc

_N = 200
_B = 10
_L = 16
_STD0 = 0.62177
_STD1 = 2.0
_LOG_STD0 = math.log(_STD0)
_LOG_STD1 = math.log(_STD1)
_HALF_LOG_2PI = 0.5 * math.log(2.0 * math.pi)
_SCALE = float(_N) / float(_B)


def _body(x_hbm, m1_hbm, ind_hbm, data_hbm, out_hbm,
          x_v, m1_v, ind_v, data_v, out_v, sem):
    cid = lax.axis_index("c")
    sid = lax.axis_index("s")

    @pl.when(jnp.logical_and(cid == 0, sid == 0))
    def _():
        c1 = pltpu.make_async_copy(x_hbm, x_v.at[pl.ds(0, 1)], sem)
        c2 = pltpu.make_async_copy(m1_hbm, m1_v.at[pl.ds(0, 1)], sem)
        c3 = pltpu.make_async_copy(ind_hbm, ind_v.at[pl.ds(0, _B)], sem)
        c4 = pltpu.make_async_copy(data_hbm, data_v, sem)
        c1.start()
        c2.start()
        c3.start()
        c4.start()
        c1.wait()
        c2.wait()
        c3.wait()
        c4.wait()

        lane = lax.iota(jnp.int32, _L)
        mask = lane < _B
        idx = jnp.where(mask, ind_v[...], jnp.zeros((_L,), jnp.int32))
        vals = plsc.load_gather(data_v, [idx])
        xs = x_v[...][0]
        m1s = m1_v[...][0]

        branch0 = m1s < 0.5
        std = jnp.where(branch0, jnp.float32(_STD0), jnp.float32(_STD1))
        log_std = jnp.where(branch0, jnp.float32(_LOG_STD0),
                            jnp.float32(_LOG_STD1))
        z = (vals - xs) / std
        lp = -0.5 * z * z - log_std - jnp.float32(_HALF_LOG_2PI)
        ll = jnp.sum(jnp.where(mask, lp, jnp.float32(0.0)))
        prior_x = -0.5 * xs * xs - jnp.float32(_HALF_LOG_2PI)
        total = prior_x + jnp.float32(_SCALE) * ll

        out_v[...] = jnp.broadcast_to(total, (_L,))
        pltpu.sync_copy(out_v.at[pl.ds(0, 1)], out_hbm)


@jax.jit
def _log_joint(x, m1, ind, data):
    mesh = plsc.VectorSubcoreMesh(core_axis_name="c", subcore_axis_name="s",
                                  num_cores=1, num_subcores=1)
    return pl.kernel(
        _body,
        out_type=jax.ShapeDtypeStruct((1,), jnp.float32),
        mesh=mesh,
        compiler_params=pltpu.CompilerParams(
            needs_layout_passes=False,
            disable_bounds_checks=True,
            disable_semaphore_checks=True,
            skip_device_barrier=True,
        ),
        scratch_types=[
            pltpu.VMEM((_L,), jnp.float32),
            pltpu.VMEM((_L,), jnp.float32),
            pltpu.VMEM((_L,), jnp.int32),
            pltpu.VMEM((_N,), jnp.float32),
            pltpu.VMEM((_L,), jnp.float32),
            pltpu.SemaphoreType.DMA,
        ],
    )(x, m1, ind, data)


def kernel(x, m1, ind, data):
    return _log_joint(x, m1, ind, data)[0]


# speedup vs baseline: 1.2478x; 1.0945x over previous
import functools
import math

import jax
import jax.numpy as jnp
from jax import lax
from jax.experimental import pallas as pl
from jax.experimental.pallas import tpu as pltpu
from jax.experimental.pallas import tpu_sc as plsc

_N = 200
_B = 10
_STD0 = 0.62177
_STD1 = 2.0
_LOG_STD0 = math.log(_STD0)
_LOG_STD1 = math.log(_STD1)
_HALF_LOG_2PI = 0.5 * math.log(2.0 * math.pi)
_SCALE = float(_N) / float(_B)


def _body(x_hbm, m1_hbm, ind_hbm, data_hbm, out_hbm,
          x_s, m1_s, ind_s, data_s, out_s, sem):
    c1 = pltpu.make_async_copy(x_hbm, x_s, sem)
    c2 = pltpu.make_async_copy(m1_hbm, m1_s, sem)
    c3 = pltpu.make_async_copy(ind_hbm, ind_s, sem)
    c4 = pltpu.make_async_copy(data_hbm, data_s, sem)
    c1.start()
    c2.start()
    c3.start()
    c4.start()
    c1.wait()
    c2.wait()
    c3.wait()
    c4.wait()

    xs = x_s[0]
    m1s = m1_s[0]
    branch0 = m1s < 0.5
    inv_var = jnp.where(branch0, jnp.float32(1.0 / (_STD0 * _STD0)),
                        jnp.float32(1.0 / (_STD1 * _STD1)))
    const = jnp.where(
        branch0,
        jnp.float32(-_B * (_LOG_STD0 + _HALF_LOG_2PI)),
        jnp.float32(-_B * (_LOG_STD1 + _HALF_LOG_2PI)))

    def step(i, acc):
        d = data_s[ind_s[i]] - xs
        return acc + d * d

    ss = lax.fori_loop(0, _B, step, jnp.float32(0.0))
    ll = const - 0.5 * ss * inv_var
    prior_x = -0.5 * xs * xs - jnp.float32(_HALF_LOG_2PI)
    out_s[0] = prior_x + jnp.float32(_SCALE) * ll
    pltpu.sync_copy(out_s, out_hbm)


@jax.jit
def _log_joint(x, m1, ind, data):
    mesh = plsc.ScalarSubcoreMesh(axis_name="c", num_cores=1)
    return pl.kernel(
        _body,
        out_type=jax.ShapeDtypeStruct((1,), jnp.float32),
        mesh=mesh,
        compiler_params=pltpu.CompilerParams(
            needs_layout_passes=False,
            disable_bounds_checks=True,
            disable_semaphore_checks=True,
            skip_device_barrier=True,
        ),
        scratch_types=[
            pltpu.SMEM((1,), jnp.float32),
            pltpu.SMEM((1,), jnp.float32),
            pltpu.SMEM((_B,), jnp.int32),
            pltpu.SMEM((_N,), jnp.float32),
            pltpu.SMEM((1,), jnp.float32),
            pltpu.SemaphoreType.DMA,
        ],
    )(x, m1, ind, data)


def kernel(x, m1, ind, data):
    return _log_joint(x, m1, ind, data)[0]
